# Initial kernel scaffold; baseline (speedup 1.0000x reference)
import jax
import jax.numpy as jnp
from jax.experimental import pallas as pl


def kernel(source_points, target_points):
    raise NotImplementedError("write your pallas kernel here")


# speedup vs baseline: 1.1793x; 1.1793x over previous
import jax
import jax.numpy as jnp
from jax.experimental import pallas as pl

_B, _N, _TILE = 8, 2048, 256
_INF = float("inf")


def _f_sum(val2d):
    d2 = jnp.maximum(val2d, 0.0)
    return jnp.sum(jnp.sqrt(1.0 + d2) + jnp.sqrt(d2), axis=(0, 1), keepdims=True)


def _pga_loss_kernel(s_ref, tt_ref, row_ref, col_ref):
    b = pl.program_id(0)

    @pl.when(b == 0)
    def _init():
        row_ref[...] = jnp.zeros((1, 1), jnp.float32)
        col_ref[...] = jnp.zeros((1, 1), jnp.float32)

    s = s_ref[0]
    tt = tt_ref[0]
    ttb = tt.astype(jnp.bfloat16)
    tn = jnp.sum(tt * tt, axis=0, keepdims=True)

    rowsum = jnp.zeros((1, 1), dtype=jnp.float32)
    colmin = jnp.full((1, _N), _INF, dtype=jnp.float32)
    colval = jnp.full((1, _N), _INF, dtype=jnp.float32)
    for i in range(_N // _TILE):
        st = s[i * _TILE:(i + 1) * _TILE, :]
        sn = jnp.sum(st * st, axis=1, keepdims=True)
        base = sn + tn
        g_bf = jnp.dot(st.astype(jnp.bfloat16), ttb,
                       preferred_element_type=jnp.float32)
        d_bf = base - 2.0 * g_bf
        g_hi = jnp.dot(st, tt, preferred_element_type=jnp.float32,
                       precision=jax.lax.Precision.HIGHEST)
        d_hi = base - 2.0 * g_hi

        rmin = jnp.min(d_bf, axis=1, keepdims=True)
        rowval = jnp.min(jnp.where(d_bf == rmin, d_hi, _INF),
                         axis=1, keepdims=True)
        rowsum = rowsum + _f_sum(rowval)

        cmin_t = jnp.min(d_bf, axis=0, keepdims=True)
        cval_t = jnp.min(jnp.where(d_bf == cmin_t, d_hi, _INF),
                         axis=0, keepdims=True)
        better = cmin_t < colmin
        equal = cmin_t == colmin
        colval = jnp.where(better, cval_t,
                           jnp.where(equal, jnp.minimum(colval, cval_t),
                                     colval))
        colmin = jnp.minimum(colmin, cmin_t)

    colsum = _f_sum(colval)
    row_ref[...] += rowsum
    col_ref[...] += colsum


def kernel(source_points, target_points):
    tt = jnp.swapaxes(target_points, 1, 2)
    row, col = pl.pallas_call(
        _pga_loss_kernel,
        grid=(_B,),
        in_specs=[
            pl.BlockSpec((1, _N, 3), lambda b: (b, 0, 0)),
            pl.BlockSpec((1, 3, _N), lambda b: (b, 0, 0)),
        ],
        out_specs=[
            pl.BlockSpec((1, 1), lambda b: (0, 0)),
            pl.BlockSpec((1, 1), lambda b: (0, 0)),
        ],
        out_shape=[
            jax.ShapeDtypeStruct((1, 1), jnp.float32),
            jax.ShapeDtypeStruct((1, 1), jnp.float32),
        ],
    )(source_points, tt)

    total = jnp.float32(_B * _N)
    l_t2s = row[0, 0] / total
    l_s2t = col[0, 0] / total

    def _norm(loss):
        return jnp.clip((loss - 1.0) * 0.5, 0.0, 1.0)

    return 0.5 * (_norm(l_t2s) + _norm(l_s2t))
